# baseline (speedup 1.0000x reference)
import functools

import jax
import jax.numpy as jnp
from jax import lax
from jax.experimental import pallas as pl
from jax.experimental.pallas import tpu as pltpu
from jax.experimental.pallas import tpu_sc as plsc

D_MODEL = 64
SCALE = 8.0

NC = 2
NS = 16
NW = NC * NS

B_TOTAL = 4096 * 200
B_PER_W = B_TOTAL // NW
CHUNK = 512
KSUB = CHUNK // 128
NCHUNK = B_PER_W // CHUNK


def _body(idx_hbm, w_hbm, out_hbm, idx_v, rows_v, gsem):
    c = lax.axis_index("c")
    s = lax.axis_index("s")
    wid = s * NC + c

    def chunk_body(ci, carry):
        pltpu.sync_copy(idx_hbm.at[wid, ci], idx_v)
        for j in range(KSUB):
            pltpu.async_copy(
                w_hbm.at[idx_v.at[j]],
                rows_v.at[pl.ds(j * 128, 128)],
                gsem,
            ).wait()

        def row_body(r, rcarry):
            for j in range(D_MODEL // 16):
                sl = pl.ds(j * 16, 16)
                rows_v[r, sl] = rows_v[r, sl] * SCALE
            return rcarry

        lax.fori_loop(0, CHUNK, row_body, 0, unroll=4)

        pltpu.sync_copy(rows_v, out_hbm.at[wid, ci])
        return carry

    lax.fori_loop(0, NCHUNK, chunk_body, 0)


@jax.jit
def kernel(x, W):
    idx = x.reshape(-1).astype(jnp.int32).reshape(NW, NCHUNK, KSUB, 128)
    mesh = plsc.VectorSubcoreMesh(core_axis_name="c", subcore_axis_name="s")
    out = pl.kernel(
        _body,
        out_type=jax.ShapeDtypeStruct((NW, NCHUNK, CHUNK, D_MODEL), jnp.float32),
        mesh=mesh,
        compiler_params=pltpu.CompilerParams(use_tc_tiling_on_sc=False),
        scratch_types=[
            pltpu.VMEM((KSUB, 128), jnp.int32),
            pltpu.VMEM((CHUNK, D_MODEL), jnp.float32),
            pltpu.SemaphoreType.DMA,
        ],
    )(idx, W)
    return out.reshape(x.shape[0], x.shape[1], D_MODEL)


# speedup vs baseline: 1.1586x; 1.1586x over previous
import jax
import jax.numpy as jnp
from jax import lax
from jax.experimental import pallas as pl
from jax.experimental.pallas import tpu as pltpu
from jax.experimental.pallas import tpu_sc as plsc

D_MODEL = 64
SCALE = 8.0

NC = 2
NS = 16
NW = NC * NS

B_TOTAL = 4096 * 200
B_PER_W = B_TOTAL // NW
CHUNK = 512
KSUB = CHUNK // 128
NCHUNK = B_PER_W // CHUNK
NSLICE = KSUB * NCHUNK


def _start_gather(w_hbm, idx_v, rows_v, gsem, ci, b):
    for j in range(KSUB):
        pltpu.async_copy(
            w_hbm.at[idx_v.at[ci * KSUB + j]],
            rows_v.at[b, pl.ds(j * 128, 128)],
            gsem.at[b],
        )


def _wait_gather(w_hbm, idx_v, rows_v, gsem, b):
    for j in range(KSUB):
        pltpu.make_async_copy(
            w_hbm.at[idx_v.at[j]],
            rows_v.at[b, pl.ds(j * 128, 128)],
            gsem.at[b],
        ).wait()


def _body(idx_hbm, w_hbm, out_hbm, idx_v, rows_v, isem, gsem, ssem):
    c = lax.axis_index("c")
    s = lax.axis_index("s")
    wid = s * NC + c

    pltpu.async_copy(idx_hbm.at[wid], idx_v, isem).wait()

    _start_gather(w_hbm, idx_v, rows_v, gsem, 0, 0)

    def chunk_body(ci, carry):
        b = lax.rem(ci, 2)
        nb = 1 - b
        _wait_gather(w_hbm, idx_v, rows_v, gsem, b)

        @pl.when(ci + 1 < NCHUNK)
        def _():
            @pl.when(ci >= 1)
            def _():
                pltpu.make_async_copy(
                    rows_v.at[nb], out_hbm.at[wid, 0], ssem.at[nb]
                ).wait()

            _start_gather(w_hbm, idx_v, rows_v, gsem, ci + 1, nb)

        def row_body(r, rcarry):
            for j in range(D_MODEL // 16):
                sl = pl.ds(j * 16, 16)
                rows_v[b, r, sl] = rows_v[b, r, sl] * SCALE
            return rcarry

        lax.fori_loop(0, CHUNK, row_body, 0, unroll=8)

        pltpu.async_copy(rows_v.at[b], out_hbm.at[wid, ci], ssem.at[b])
        return carry

    lax.fori_loop(0, NCHUNK, chunk_body, 0)

    for b in range(2):
        pltpu.make_async_copy(
            rows_v.at[b], out_hbm.at[wid, 0], ssem.at[b]
        ).wait()


@jax.jit
def kernel(x, W):
    idx = x.reshape(-1).astype(jnp.int32).reshape(NW, NSLICE, 128)
    mesh = plsc.VectorSubcoreMesh(core_axis_name="c", subcore_axis_name="s")
    out = pl.kernel(
        _body,
        out_type=jax.ShapeDtypeStruct((NW, NCHUNK, CHUNK, D_MODEL), jnp.float32),
        mesh=mesh,
        compiler_params=pltpu.CompilerParams(use_tc_tiling_on_sc=False),
        scratch_types=[
            pltpu.VMEM((NSLICE, 128), jnp.int32),
            pltpu.VMEM((2, CHUNK, D_MODEL), jnp.float32),
            pltpu.SemaphoreType.DMA,
            pltpu.SemaphoreType.DMA((2,)),
            pltpu.SemaphoreType.DMA((2,)),
        ],
    )(idx, W)
    return out.reshape(x.shape[0], x.shape[1], D_MODEL)
